# baseline (device time: 31872 ns/iter reference)
import jax
import jax.numpy as jnp
from jax import lax
from jax.experimental import pallas as pl
from jax.experimental.pallas import tpu as pltpu

N_DEV = 4


def kernel(x, w_mat):
    k_glob, k_per = x.shape
    m_per = k_glob // N_DEV
    n = w_mat.shape[1]
    bf16 = jnp.bfloat16

    def body(x_hbm, w_hbm, out_hbm, x_blk, w_vmem, w_bf, recv_ref, send_ref,
             out_vmem, blk_sems, local_sems, send_sems, recv_sems):
        my_pos = lax.axis_index("i")

        def w_slice(dev):
            return w_bf[pl.ds(dev * k_per, k_per), :]

        with jax.named_scope("fetch_start"):
            x_cps = {}
            for off in (1, 3, 2, 0):
                src_dev = (my_pos + off) % N_DEV
                slot = (off - 1) % N_DEV
                cp = pltpu.make_async_copy(
                    x_hbm.at[pl.ds(src_dev * m_per, m_per), :],
                    x_blk.at[slot],
                    blk_sems.at[slot],
                )
                cp.start()
                x_cps[off] = cp
            w_cp = pltpu.make_async_copy(w_hbm, w_vmem, local_sems.at[1])
            w_cp.start()

        with jax.named_scope("barrier"):
            barrier_sem = pltpu.get_barrier_semaphore()
            for off in range(1, N_DEV):
                pl.semaphore_signal(
                    barrier_sem, inc=1,
                    device_id=((my_pos + off) % N_DEV,),
                    device_id_type=pl.DeviceIdType.MESH,
                )
            pl.semaphore_wait(barrier_sem, N_DEV - 1)

        def make_rdma(off):
            dst = (my_pos + off) % N_DEV
            return pltpu.make_async_remote_copy(
                src_ref=send_ref.at[off - 1],
                dst_ref=recv_ref.at[off - 1],
                send_sem=send_sems.at[off - 1],
                recv_sem=recv_sems.at[off - 1],
                device_id=(dst,),
                device_id_type=pl.DeviceIdType.MESH,
            )

        with jax.named_scope("send_wave1"):
            rdmas = {}
            for off in (1, 3):
                x_cps[off].wait()
                send_ref[off - 1] = x_blk[off - 1].astype(bf16)
                rdmas[off] = make_rdma(off)
                rdmas[off].start()

        with jax.named_scope("stage_diag"):
            x_cps[2].wait()
            send_ref[1] = x_blk[1].astype(bf16)

        with jax.named_scope("send_wave2"):
            rdmas[1].wait_send()
            rdmas[3].wait_send()
            rdmas[2] = make_rdma(2)
            rdmas[2].start()

        with jax.named_scope("w_cast"):
            w_cp.wait()
            w_bf[:, :] = w_vmem[:, :].astype(bf16)

        with jax.named_scope("dot_local"):
            x_cps[0].wait()
            acc = jnp.dot(
                x_blk[3].astype(bf16),
                w_slice(my_pos),
                preferred_element_type=jnp.float32,
            )

        for off in (1, 3):
            with jax.named_scope(f"recv_dot_{off}"):
                src_dev = (my_pos - off) % N_DEV
                rdmas[off].wait_recv()
                acc = acc + jnp.dot(
                    recv_ref[off - 1],
                    w_slice(src_dev),
                    preferred_element_type=jnp.float32,
                )

        with jax.named_scope("recv_diag_epilogue"):
            src2 = (my_pos - 2) % N_DEV
            rdmas[2].wait_recv()
            half = n // 2
            out_cps = []
            for h in range(2):
                cols = pl.ds(h * half, half)
                part = acc[:, h * half:(h + 1) * half] + jnp.dot(
                    recv_ref[1],
                    w_slice(src2)[:, h * half:(h + 1) * half],
                    preferred_element_type=jnp.float32,
                )
                out_vmem[:, cols] = jnp.maximum(part, 0.0).astype(bf16)
                cp = pltpu.make_async_copy(
                    out_vmem.at[:, cols], out_hbm.at[:, cols],
                    local_sems.at[h],
                )
                cp.start()
                out_cps.append(cp)
            for cp in out_cps:
                cp.wait()

        with jax.named_scope("drain_send"):
            rdmas[2].wait_send()

    return pl.pallas_call(
        body,
        out_shape=jax.ShapeDtypeStruct((m_per, n), bf16),
        in_specs=[
            pl.BlockSpec(memory_space=pl.ANY),
            pl.BlockSpec(memory_space=pl.ANY),
        ],
        out_specs=pl.BlockSpec(memory_space=pl.ANY),
        scratch_shapes=[
            pltpu.VMEM((N_DEV, m_per, k_per), jnp.float32),
            pltpu.VMEM((k_glob, n), jnp.float32),
            pltpu.VMEM((k_glob, n), bf16),
            pltpu.VMEM((N_DEV - 1, m_per, k_per), bf16),
            pltpu.VMEM((N_DEV - 1, m_per, k_per), bf16),
            pltpu.VMEM((m_per, n), bf16),
            pltpu.SemaphoreType.DMA((N_DEV,)),
            pltpu.SemaphoreType.DMA((2,)),
            pltpu.SemaphoreType.DMA((N_DEV - 1,)),
            pltpu.SemaphoreType.DMA((N_DEV - 1,)),
        ],
        compiler_params=pltpu.CompilerParams(
            collective_id=0, vmem_limit_bytes=100 * 1024 * 1024
        ),
    )(x, w_mat)


# device time: 24323 ns/iter; 1.3104x vs baseline; 1.3104x over previous
import jax
import jax.numpy as jnp
from jax import lax
from jax.experimental import pallas as pl
from jax.experimental.pallas import tpu as pltpu

N_DEV = 4


def kernel(x, w_mat):
    k_glob, k_per = x.shape
    m_per = k_glob // N_DEV
    n = w_mat.shape[1]
    bf16 = jnp.bfloat16

    def body(x_hbm, w_hbm, out_hbm, x_vmem, w_vmem, w_bf, recv_ref, send_ref,
             out_vmem, local_sems, send_sems, recv_sems):
        my_pos = lax.axis_index("i")

        def w_slice(dev):
            return w_bf[pl.ds(dev * k_per, k_per), :]

        with jax.named_scope("fetch_start"):
            x_cp = pltpu.make_async_copy(x_hbm, x_vmem, local_sems.at[0])
            x_cp.start()
            w_cp = pltpu.make_async_copy(w_hbm, w_vmem, local_sems.at[1])
            w_cp.start()

        with jax.named_scope("barrier"):
            barrier_sem = pltpu.get_barrier_semaphore()
            for off in range(1, N_DEV):
                pl.semaphore_signal(
                    barrier_sem, inc=1,
                    device_id=((my_pos + off) % N_DEV,),
                    device_id_type=pl.DeviceIdType.MESH,
                )
            pl.semaphore_wait(barrier_sem, N_DEV - 1)

        def make_rdma(off):
            dst = (my_pos + off) % N_DEV
            return pltpu.make_async_remote_copy(
                src_ref=send_ref.at[off - 1],
                dst_ref=recv_ref.at[off - 1],
                send_sem=send_sems.at[off - 1],
                recv_sem=recv_sems.at[off - 1],
                device_id=(dst,),
                device_id_type=pl.DeviceIdType.MESH,
            )

        with jax.named_scope("send_wave1"):
            x_cp.wait()
            rdmas = {}
            for off in (1, 3):
                dst = (my_pos + off) % N_DEV
                send_ref[off - 1] = (
                    x_vmem[pl.ds(dst * m_per, m_per), :].astype(bf16)
                )
                rdmas[off] = make_rdma(off)
                rdmas[off].start()

        with jax.named_scope("stage_diag"):
            dst2 = (my_pos + 2) % N_DEV
            send_ref[1] = x_vmem[pl.ds(dst2 * m_per, m_per), :].astype(bf16)

        with jax.named_scope("send_wave2"):
            rdmas[1].wait_send()
            rdmas[3].wait_send()
            rdmas[2] = make_rdma(2)
            rdmas[2].start()

        with jax.named_scope("w_cast"):
            w_cp.wait()
            w_bf[:, :] = w_vmem[:, :].astype(bf16)

        with jax.named_scope("dot_local"):
            acc = jnp.dot(
                x_vmem[pl.ds(my_pos * m_per, m_per), :].astype(bf16),
                w_slice(my_pos),
                preferred_element_type=jnp.float32,
            )

        for off in (1, 3):
            with jax.named_scope(f"recv_dot_{off}"):
                src_dev = (my_pos - off) % N_DEV
                rdmas[off].wait_recv()
                acc = acc + jnp.dot(
                    recv_ref[off - 1],
                    w_slice(src_dev),
                    preferred_element_type=jnp.float32,
                )

        with jax.named_scope("recv_diag_epilogue"):
            src2 = (my_pos - 2) % N_DEV
            rdmas[2].wait_recv()
            half = n // 2
            out_cps = []
            for h in range(2):
                cols = pl.ds(h * half, half)
                part = acc[:, h * half:(h + 1) * half] + jnp.dot(
                    recv_ref[1],
                    w_slice(src2)[:, h * half:(h + 1) * half],
                    preferred_element_type=jnp.float32,
                )
                out_vmem[:, cols] = jnp.maximum(part, 0.0).astype(bf16)
                cp = pltpu.make_async_copy(
                    out_vmem.at[:, cols], out_hbm.at[:, cols],
                    local_sems.at[h],
                )
                cp.start()
                out_cps.append(cp)
            for cp in out_cps:
                cp.wait()

        with jax.named_scope("drain_send"):
            rdmas[2].wait_send()

    return pl.pallas_call(
        body,
        out_shape=jax.ShapeDtypeStruct((m_per, n), bf16),
        in_specs=[
            pl.BlockSpec(memory_space=pl.ANY),
            pl.BlockSpec(memory_space=pl.ANY),
        ],
        out_specs=pl.BlockSpec(memory_space=pl.ANY),
        scratch_shapes=[
            pltpu.VMEM((k_glob, k_per), jnp.float32),
            pltpu.VMEM((k_glob, n), jnp.float32),
            pltpu.VMEM((k_glob, n), bf16),
            pltpu.VMEM((N_DEV - 1, m_per, k_per), bf16),
            pltpu.VMEM((N_DEV - 1, m_per, k_per), bf16),
            pltpu.VMEM((m_per, n), bf16),
            pltpu.SemaphoreType.DMA((2,)),
            pltpu.SemaphoreType.DMA((N_DEV - 1,)),
            pltpu.SemaphoreType.DMA((N_DEV - 1,)),
        ],
        compiler_params=pltpu.CompilerParams(
            collective_id=0, vmem_limit_bytes=100 * 1024 * 1024
        ),
    )(x, w_mat)
